# baseline (speedup 1.0000x reference)
import functools

import jax
import jax.numpy as jnp
from jax import lax
from jax.experimental import pallas as pl
from jax.experimental.pallas import tpu as pltpu
from jax.experimental.pallas import tpu_sc as plsc

_CHUNK = 128
_NBUF = 6
_HEAD = (32, 32, 64)


@functools.lru_cache(maxsize=None)
def _make_embed_kernel(V, D, bsz, seq_len, idx_dtype):
    info = plsc.get_sparse_core_info()
    NC, NS, L = info.num_cores, info.num_subcores, info.num_lanes
    NW = NC * NS
    B = bsz * seq_len
    assert B % (NW * _CHUNK) == 0 and D % L == 0
    b_per_w = B // NW
    assert seq_len % b_per_w == 0
    sizes = list(_HEAD) + [_CHUNK] * ((b_per_w - sum(_HEAD)) // _CHUNK)
    assert sum(sizes) == b_per_w and all(s % 8 == 0 for s in sizes)
    offs = [sum(sizes[:k]) for k in range(len(sizes))]
    n = len(sizes)
    head_idx = sum(_HEAD)
    mesh = plsc.VectorSubcoreMesh(core_axis_name="c", subcore_axis_name="s")

    @functools.partial(
        pl.kernel,
        mesh=mesh,
        out_type=[
            jax.ShapeDtypeStruct((bsz, seq_len, D), jnp.float32),
            jax.ShapeDtypeStruct((bsz, seq_len), idx_dtype),
        ],
        scratch_types=(
            [pltpu.VMEM((b_per_w,), jnp.int32),
             pltpu.VMEM((b_per_w,), jnp.int32)]
            + [pltpu.VMEM((_CHUNK, D), jnp.float32) for _ in range(_NBUF)]
            + [pltpu.SemaphoreType.DMA for _ in range(2 * _NBUF + 2)]
        ),
    )
    def embed_kernel(table_hbm, ids_hbm, out_hbm, pos_hbm,
                     idx_v, pos_v, *bufs_and_sems):
        bufs = bufs_and_sems[:_NBUF]
        gsems = bufs_and_sems[_NBUF:2 * _NBUF]
        wsems = bufs_and_sems[2 * _NBUF:3 * _NBUF]
        isem_a = bufs_and_sems[3 * _NBUF]
        isem_b = bufs_and_sems[3 * _NBUF + 1]
        wid = lax.axis_index("s") * NC + lax.axis_index("c")
        base = wid * b_per_w
        row = base // seq_len
        col = base % seq_len
        idx_a = pltpu.async_copy(ids_hbm.at[row, pl.ds(col, head_idx)],
                                 idx_v.at[pl.ds(0, head_idx)], isem_a)
        idx_b = pltpu.async_copy(
            ids_hbm.at[row, pl.ds(col + head_idx, b_per_w - head_idx)],
            idx_v.at[pl.ds(head_idx, b_per_w - head_idx)], isem_b)

        def start_gather(k):
            return pltpu.async_copy(
                table_hbm.at[idx_v.at[pl.ds(offs[k], sizes[k])]],
                bufs[k % _NBUF].at[pl.ds(0, sizes[k])], gsems[k % _NBUF])

        def start_writeback(k):
            return pltpu.async_copy(
                bufs[k % _NBUF].at[pl.ds(0, sizes[k])],
                out_hbm.at[row, pl.ds(col + offs[k], sizes[k])],
                wsems[k % _NBUF])

        gathers = [None] * n
        writebacks = [None] * n
        idx_a.wait()
        for k in range(min(_NBUF, n)):
            if k == len(_HEAD):
                idx_b.wait()
            gathers[k] = start_gather(k)
        if _NBUF <= len(_HEAD):
            idx_b.wait()

        for j in range(b_per_w // L):
            pos_v[pl.ds(j * L, L)] = lax.iota(jnp.int32, L) + (col + j * L)
        pltpu.sync_copy(pos_v, pos_hbm.at[row, pl.ds(col, b_per_w)])

        for k in range(n):
            gathers[k].wait()
            writebacks[k] = start_writeback(k)
            nxt = k - 1 + _NBUF
            if k >= 1 and nxt < n:
                writebacks[k - 1].wait()
                gathers[nxt] = start_gather(nxt)
        for k in range(max(0, n - _NBUF), n):
            writebacks[k].wait()

    return embed_kernel


def kernel(input_ids, embed_table):
    bsz, seq_len = input_ids.shape
    V, D = embed_table.shape
    ids = input_ids.astype(jnp.int32)
    fn = _make_embed_kernel(V, D, bsz, seq_len, jnp.dtype(input_ids.dtype))
    hidden, position_ids = fn(embed_table, ids)
    return (hidden, position_ids)


# speedup vs baseline: 1.0049x; 1.0049x over previous
import functools

import jax
import jax.numpy as jnp
from jax import lax
from jax.experimental import pallas as pl
from jax.experimental.pallas import tpu as pltpu
from jax.experimental.pallas import tpu_sc as plsc

_CHUNK = 128
_NBUF = 6
_TAIL = (64, 32, 32)


@functools.lru_cache(maxsize=None)
def _make_embed_kernel(V, D, bsz, seq_len, idx_dtype):
    info = plsc.get_sparse_core_info()
    NC, NS, L = info.num_cores, info.num_subcores, info.num_lanes
    NW = NC * NS
    B = bsz * seq_len
    assert B % (NW * _CHUNK) == 0 and D % L == 0
    b_per_w = B // NW
    assert seq_len % b_per_w == 0
    sizes = [_CHUNK] * ((b_per_w - sum(_TAIL)) // _CHUNK) + list(_TAIL)
    assert sum(sizes) == b_per_w and all(s % 8 == 0 for s in sizes)
    offs = [sum(sizes[:k]) for k in range(len(sizes))]
    n = len(sizes)
    mesh = plsc.VectorSubcoreMesh(core_axis_name="c", subcore_axis_name="s")

    @functools.partial(
        pl.kernel,
        mesh=mesh,
        out_type=[
            jax.ShapeDtypeStruct((bsz, seq_len, D), jnp.float32),
            jax.ShapeDtypeStruct((bsz, seq_len), idx_dtype),
        ],
        scratch_types=(
            [pltpu.VMEM((b_per_w,), jnp.int32),
             pltpu.VMEM((b_per_w,), jnp.int32)]
            + [pltpu.VMEM((_CHUNK, D), jnp.float32) for _ in range(_NBUF)]
            + [pltpu.SemaphoreType.DMA for _ in range(2 * _NBUF + 1)]
        ),
    )
    def embed_kernel(table_hbm, ids_hbm, out_hbm, pos_hbm,
                     idx_v, pos_v, *bufs_and_sems):
        bufs = bufs_and_sems[:_NBUF]
        gsems = bufs_and_sems[_NBUF:2 * _NBUF]
        wsems = bufs_and_sems[2 * _NBUF:3 * _NBUF]
        isem = bufs_and_sems[3 * _NBUF]
        wid = lax.axis_index("s") * NC + lax.axis_index("c")
        base = wid * b_per_w
        row = base // seq_len
        col = base % seq_len
        pltpu.async_copy(ids_hbm.at[row, pl.ds(col, b_per_w)], idx_v,
                         isem).wait()

        def start_gather(k):
            return pltpu.async_copy(
                table_hbm.at[idx_v.at[pl.ds(offs[k], sizes[k])]],
                bufs[k % _NBUF].at[pl.ds(0, sizes[k])], gsems[k % _NBUF])

        def start_writeback(k):
            return pltpu.async_copy(
                bufs[k % _NBUF].at[pl.ds(0, sizes[k])],
                out_hbm.at[row, pl.ds(col + offs[k], sizes[k])],
                wsems[k % _NBUF])

        gathers = [None] * n
        writebacks = [None] * n
        for k in range(min(_NBUF, n)):
            gathers[k] = start_gather(k)

        for j in range(b_per_w // L):
            pos_v[pl.ds(j * L, L)] = lax.iota(jnp.int32, L) + (col + j * L)
        pltpu.sync_copy(pos_v, pos_hbm.at[row, pl.ds(col, b_per_w)])

        for k in range(n):
            gathers[k].wait()
            writebacks[k] = start_writeback(k)
            nxt = k - 1 + _NBUF
            if k >= 1 and nxt < n:
                writebacks[k - 1].wait()
                gathers[nxt] = start_gather(nxt)
        for k in range(max(0, n - _NBUF), n):
            writebacks[k].wait()

    return embed_kernel


def kernel(input_ids, embed_table):
    bsz, seq_len = input_ids.shape
    V, D = embed_table.shape
    ids = input_ids.astype(jnp.int32)
    fn = _make_embed_kernel(V, D, bsz, seq_len, jnp.dtype(input_ids.dtype))
    hidden, position_ids = fn(embed_table, ids)
    return (hidden, position_ids)


# speedup vs baseline: 1.0131x; 1.0081x over previous
import functools

import jax
import jax.numpy as jnp
from jax import lax
from jax.experimental import pallas as pl
from jax.experimental.pallas import tpu as pltpu
from jax.experimental.pallas import tpu_sc as plsc

_CHUNK = 128
_NBUF = 6


@functools.lru_cache(maxsize=None)
def _make_embed_kernel(V, D, bsz, seq_len, idx_dtype):
    info = plsc.get_sparse_core_info()
    NC, NS, L = info.num_cores, info.num_subcores, info.num_lanes
    NW = NC * NS
    B = bsz * seq_len
    assert B % (NW * _CHUNK) == 0 and D % L == 0
    b_per_w = B // NW
    n_chunks = b_per_w // _CHUNK
    assert seq_len % b_per_w == 0
    mesh = plsc.VectorSubcoreMesh(core_axis_name="c", subcore_axis_name="s")

    @functools.partial(
        pl.kernel,
        mesh=mesh,
        out_type=[
            jax.ShapeDtypeStruct((bsz, seq_len, D), jnp.float32),
            jax.ShapeDtypeStruct((bsz, seq_len), idx_dtype),
        ],
        scratch_types=(
            [pltpu.VMEM((b_per_w,), jnp.int32),
             pltpu.VMEM((b_per_w,), jnp.int32)]
            + [pltpu.VMEM((_CHUNK, D), jnp.float32) for _ in range(_NBUF)]
            + [pltpu.SemaphoreType.DMA for _ in range(2 * _NBUF + 1)]
        ),
    )
    def embed_kernel(table_hbm, ids_hbm, out_hbm, pos_hbm,
                     idx_v, pos_v, *bufs_and_sems):
        bufs = bufs_and_sems[:_NBUF]
        gsems = bufs_and_sems[_NBUF:2 * _NBUF]
        wsems = bufs_and_sems[2 * _NBUF:3 * _NBUF]
        isem = bufs_and_sems[3 * _NBUF]
        wid = lax.axis_index("s") * NC + lax.axis_index("c")
        base = wid * b_per_w
        row = base // seq_len
        col = base % seq_len
        pltpu.async_copy(ids_hbm.at[row, pl.ds(col, b_per_w)], idx_v,
                         isem).wait()

        def start_gather(j):
            return pltpu.async_copy(
                table_hbm.at[idx_v.at[pl.ds(j * _CHUNK, _CHUNK)]],
                bufs[j % _NBUF], gsems[j % _NBUF])

        def start_writeback(j):
            return pltpu.async_copy(
                bufs[j % _NBUF],
                out_hbm.at[row, pl.ds(col + j * _CHUNK, _CHUNK)],
                wsems[j % _NBUF])

        gathers = [None] * n_chunks
        writebacks = [None] * n_chunks
        for j in range(min(_NBUF, n_chunks)):
            gathers[j] = start_gather(j)

        for j in range(b_per_w // L):
            pos_v[pl.ds(j * L, L)] = lax.iota(jnp.int32, L) + (col + j * L)
        pltpu.sync_copy(pos_v, pos_hbm.at[row, pl.ds(col, b_per_w)])

        for j in range(n_chunks):
            gathers[j].wait()
            writebacks[j] = start_writeback(j)
            nxt = j - 1 + _NBUF
            if j >= 1 and nxt < n_chunks:
                writebacks[j - 1].wait()
                gathers[nxt] = start_gather(nxt)
        for j in range(max(0, n_chunks - _NBUF), n_chunks):
            writebacks[j].wait()

    return embed_kernel


def kernel(input_ids, embed_table):
    bsz, seq_len = input_ids.shape
    V, D = embed_table.shape
    ids = input_ids.astype(jnp.int32)
    fn = _make_embed_kernel(V, D, bsz, seq_len, jnp.dtype(input_ids.dtype))
    hidden, position_ids = fn(embed_table, ids)
    return (hidden, position_ids)
